# Initial kernel scaffold; baseline (speedup 1.0000x reference)
import jax
import jax.numpy as jnp
from jax.experimental import pallas as pl


def kernel(input_ids, weight, token_type_embeddings, position_embeddings, ln_gamma, ln_beta):
    raise NotImplementedError("write your pallas kernel here")


# speedup vs baseline: 1.8289x; 1.8289x over previous
import jax
import jax.numpy as jnp
from jax import lax
from jax.experimental import pallas as pl
from jax.experimental.pallas import tpu as pltpu
from jax.experimental.pallas import tpu_sc as plsc

B, S, H = 1024, 200, 256
NTOK = B * S
NW = 32
TOK_PER_W = NTOK // NW
ROWS_PER_W = B // NW
CHUNK = 128
NCHUNK = TOK_PER_W // CHUNK
NJ = (S + 15) // 16
EC = H // 16
EPS = 1e-6


def _rsqrt(x):
    i = lax.bitcast_convert_type(x, jnp.int32)
    i = jnp.full(i.shape, jnp.int32(0x5F3759DF)) - lax.shift_right_arithmetic(
        i, jnp.full(i.shape, jnp.int32(1)))
    y = lax.bitcast_convert_type(i, jnp.float32)
    for _ in range(3):
        y = y * (jnp.float32(1.5) - jnp.float32(0.5) * x * y * y)
    return y


def _take(x, idx):
    return x.at[idx].get(mode="promise_in_bounds")


def _bfly_sum(x, lane):
    for d in (1, 2, 4, 8):
        x = x + _take(x, lane ^ d)
    return x


def _cumsum16(m, lane):
    c = m
    zero = jnp.zeros((16,), m.dtype)
    for d in (1, 2, 4, 8):
        g = _take(c, jnp.maximum(lane - d, 0))
        c = c + jnp.where(lane >= d, g, zero)
    return c


def _body(ids_hbm, w_hbm, p_hbm, g_hbm, b_hbm, out_hbm,
          ids_v, pos_v, wbuf, pbuf, gam_v, bet_v, sem0, sem1):
    nc = 2
    wid = lax.axis_index("s") * nc + lax.axis_index("c")
    tok0 = pl.multiple_of(wid * TOK_PER_W, CHUNK)

    pltpu.sync_copy(ids_hbm.at[pl.ds(tok0, TOK_PER_W)],
                    ids_v.at[pl.ds(0, TOK_PER_W)])
    pltpu.sync_copy(g_hbm, gam_v)
    pltpu.sync_copy(b_hbm, bet_v)

    lane = lax.iota(jnp.int32, 16)
    ones_v = jnp.ones((16,), jnp.int32)
    zeros_v = jnp.zeros((16,), jnp.int32)
    last_lane = jnp.full((16,), jnp.int32(15))

    def row_body(r, carry_unused):
        base = pl.multiple_of(r * S, 8)
        carry = zeros_v
        for j in range(NJ):
            v = ids_v[pl.ds(base + j * 16, 16)]
            m = jnp.where(v != 0, ones_v, zeros_v)
            c = _cumsum16(m, lane) + carry
            pos_v[pl.ds(base + j * 16, 16)] = c * m
            if j + 1 < NJ:
                carry = _take(c, last_lane)
        return carry_unused

    lax.fori_loop(0, ROWS_PER_W, row_body, jnp.int32(0))

    def chunk_body(g, carry_unused):
        off = pl.multiple_of(g * CHUNK, CHUNK)
        cp_w = pltpu.async_copy(w_hbm.at[ids_v.at[pl.ds(off, CHUNK)]], wbuf, sem0)
        cp_p = pltpu.async_copy(p_hbm.at[pos_v.at[pl.ds(off, CHUNK)]], pbuf, sem1)
        cp_w.wait()
        cp_p.wait()

        def tok_body(t, inner_unused):
            xs = []
            acc_s = jnp.zeros((16,), jnp.float32)
            acc_q = jnp.zeros((16,), jnp.float32)
            for e in range(EC):
                x = wbuf[t, pl.ds(e * 16, 16)] + pbuf[t, pl.ds(e * 16, 16)]
                xs.append(x)
                acc_s = acc_s + x
                acc_q = acc_q + x * x
            mean = _bfly_sum(acc_s, lane) * jnp.float32(1.0 / H)
            var = _bfly_sum(acc_q, lane) * jnp.float32(1.0 / H) - mean * mean
            rstd = _rsqrt(var + jnp.float32(EPS))
            for e in range(EC):
                y = (xs[e] - mean) * rstd
                y = y * gam_v[pl.ds(e * 16, 16)] + bet_v[pl.ds(e * 16, 16)]
                wbuf[t, pl.ds(e * 16, 16)] = y
            return inner_unused

        lax.fori_loop(0, CHUNK, tok_body, jnp.int32(0))
        pltpu.sync_copy(wbuf, out_hbm.at[pl.ds(tok0 + off, CHUNK), :])
        return carry_unused

    lax.fori_loop(0, NCHUNK, chunk_body, jnp.int32(0))


def kernel(input_ids, weight, token_type_embeddings, position_embeddings,
           ln_gamma, ln_beta):
    ids_flat = input_ids.reshape(-1)
    ptab = position_embeddings + token_type_embeddings[0][None, :]
    mesh = plsc.VectorSubcoreMesh(core_axis_name="c", subcore_axis_name="s")
    k = pl.kernel(
        _body,
        mesh=mesh,
        out_type=jax.ShapeDtypeStruct((NTOK, H), jnp.float32),
        scratch_types=[
            pltpu.VMEM((TOK_PER_W + 16,), jnp.int32),
            pltpu.VMEM((TOK_PER_W + 16,), jnp.int32),
            pltpu.VMEM((CHUNK, H), jnp.float32),
            pltpu.VMEM((CHUNK, H), jnp.float32),
            pltpu.VMEM((H,), jnp.float32),
            pltpu.VMEM((H,), jnp.float32),
            pltpu.SemaphoreType.DMA,
            pltpu.SemaphoreType.DMA,
        ],
    )
    out = k(ids_flat, weight, ptab, ln_gamma, ln_beta)
    return out.reshape(B, S, H)
